# baseline (speedup 1.0000x reference)
import functools

import jax
import jax.numpy as jnp
from jax.experimental import pallas as pl
from jax.experimental.pallas import tpu as pltpu

_N = 8192
_M = 4096
_K = 32
_EPS = 1e-3
_BIG = 1e30


def _fps_body(xx_ref, xy_ref, xz_ref, xsx_ref, xsy_ref, xsz_ref, sel_ref):
    xx = xx_ref[...]
    xy = xy_ref[...]
    xz = xz_ref[...]
    shape = xx.shape
    idx = (jax.lax.broadcasted_iota(jnp.int32, shape, 0) * 128
           + jax.lax.broadcasted_iota(jnp.int32, shape, 1))
    x0x = xsx_ref[0]
    x0y = xsy_ref[0]
    x0z = xsz_ref[0]
    mind = ((xx - x0x) ** 2 + (xz - x0z) ** 2) + (xy - x0y) ** 2
    sel_ref[0] = 0

    def body(i, mind):
        maxv = jnp.max(mind)
        nxt = jnp.min(jnp.where(mind == maxv, idx, _N))
        cx = xsx_ref[nxt]
        cy = xsy_ref[nxt]
        cz = xsz_ref[nxt]
        d = ((xx - cx) ** 2 + (xz - cz) ** 2) + (xy - cy) ** 2
        sel_ref[i] = nxt
        return jnp.minimum(mind, d)

    jax.lax.fori_loop(1, _M, body, mind)


def _fps(x_cor):
    xx = x_cor[:, 0].reshape(64, 128)
    xy = x_cor[:, 1].reshape(64, 128)
    xz = x_cor[:, 2].reshape(64, 128)
    sel = pl.pallas_call(
        _fps_body,
        in_specs=[
            pl.BlockSpec(memory_space=pltpu.VMEM),
            pl.BlockSpec(memory_space=pltpu.VMEM),
            pl.BlockSpec(memory_space=pltpu.VMEM),
            pl.BlockSpec(memory_space=pltpu.SMEM),
            pl.BlockSpec(memory_space=pltpu.SMEM),
            pl.BlockSpec(memory_space=pltpu.SMEM),
        ],
        out_shape=jax.ShapeDtypeStruct((_M,), jnp.int32),
        out_specs=pl.BlockSpec(memory_space=pltpu.SMEM),
    )(xx, xy, xz, x_cor[:, 0], x_cor[:, 1], x_cor[:, 2])
    return sel


def _knn_body(nsteps, diag, bm, nc,
              yblk_ref, xt_ref, xc_ref,
              idx_out, vx_out, vy_out, vz_out, d_out, valid_out,
              d2_ref):
    yb = yblk_ref[...]
    yx = yb[:, 0:1]
    yy = yb[:, 1:2]
    yz = yb[:, 2:3]
    xt = xt_ref[...]
    xxr = xt[0:1, :]
    xyr = xt[1:2, :]
    xzr = xt[2:3, :]
    sa = yx * yx + yy * yy + yz * yz
    sb = xxr * xxr + xyr * xyr + xzr * xzr
    mm = jnp.dot(yb, xt, preferred_element_type=jnp.float32)
    d2 = sa + sb - 2.0 * mm
    col = jax.lax.broadcasted_iota(jnp.int32, (bm, nc), 1)
    if diag:
        rowi = (pl.program_id(0) * bm
                + jax.lax.broadcasted_iota(jnp.int32, (bm, nc), 0))
        d2 = jnp.where(col == rowi, jnp.inf, d2)
    d2_ref[...] = d2

    lane = jax.lax.broadcasted_iota(jnp.int32, (bm, 64), 1)
    zf = jnp.zeros((bm, 64), jnp.float32)
    zi = jnp.zeros((bm, 64), jnp.int32)

    def step(k, accs):
        acc_i, acc_x, acc_y, acc_z = accs
        d2c = d2_ref[...]
        mv = jnp.min(d2c, axis=1, keepdims=True)
        nidx = jnp.min(jnp.where(d2c == mv, col, nc), axis=1, keepdims=True)
        ohf = (col == nidx).astype(jnp.float32)
        d2_ref[...] = d2c + ohf * _BIG
        cxyz = jnp.dot(ohf, xc_ref[...],
                       preferred_element_type=jnp.float32,
                       precision=jax.lax.Precision.HIGHEST)
        acc_i = jnp.where(lane == k, nidx, acc_i)
        acc_x = jnp.where(lane == k, cxyz[:, 0:1] - yx, acc_x)
        acc_y = jnp.where(lane == k, cxyz[:, 1:2] - yy, acc_y)
        acc_z = jnp.where(lane == k, cxyz[:, 2:3] - yz, acc_z)
        return acc_i, acc_x, acc_y, acc_z

    acc_i, acc_x, acc_y, acc_z = jax.lax.fori_loop(
        0, nsteps, step, (zi, zf, zf, zf))

    dd = jnp.sqrt(acc_x * acc_x + acc_y * acc_y + acc_z * acc_z)
    vmask = dd > _EPS
    idx_out[...] = acc_i
    vx_out[...] = jnp.where(vmask, acc_x, 0.0)
    vy_out[...] = jnp.where(vmask, acc_y, 0.0)
    vz_out[...] = jnp.where(vmask, acc_z, 0.0)
    d_out[...] = jnp.where(vmask, dd, 0.0)
    valid_out[...] = vmask.astype(jnp.int32)


def _knn(y, xt, xc, nsteps, diag, bm):
    mq = y.shape[0]
    nc = xt.shape[1]
    f32 = jnp.float32
    body = functools.partial(_knn_body, nsteps, diag, bm, nc)
    outs = pl.pallas_call(
        body,
        grid=(mq // bm,),
        in_specs=[
            pl.BlockSpec((bm, 3), lambda i: (i, 0)),
            pl.BlockSpec((3, nc), lambda i: (0, 0)),
            pl.BlockSpec((nc, 3), lambda i: (0, 0)),
        ],
        out_specs=[pl.BlockSpec((bm, 64), lambda i: (i, 0))] * 6,
        out_shape=[
            jax.ShapeDtypeStruct((mq, 64), jnp.int32),
            jax.ShapeDtypeStruct((mq, 64), f32),
            jax.ShapeDtypeStruct((mq, 64), f32),
            jax.ShapeDtypeStruct((mq, 64), f32),
            jax.ShapeDtypeStruct((mq, 64), f32),
            jax.ShapeDtypeStruct((mq, 64), jnp.int32),
        ],
        scratch_shapes=[pltpu.VMEM((bm, nc), f32)],
    )(y, xt, xc)
    return outs


def kernel(x_cor, piece_index, L):
    del piece_index, L
    sel = _fps(x_cor)
    node_dst_idx = jnp.sort(sel)
    y = x_cor[node_dst_idx]
    xt = x_cor.T
    yt = y.T

    kb = _K + 1
    nbr_i, vx, vy, vz, dpad, validpad = _knn(y, xt, x_cor, kb, False, 256)
    x_idx = nbr_i[:, :kb].reshape(-1)
    y_idx = jnp.repeat(jnp.arange(_M), kb)
    edge_vec = jnp.stack(
        [vx[:, :kb], vy[:, :kb], vz[:, :kb]], axis=-1).reshape(-1, 3)
    d = dpad[:, :kb].reshape(-1)
    valid = validpad[:, :kb].reshape(-1).astype(bool)
    bi_e = jnp.stack([x_idx, y_idx])

    ci, cvx, cvy, cvz, cdp, cvp = _knn(y, yt, y, _K, True, 256)
    src = ci[:, :_K].reshape(-1)
    dst = jnp.repeat(jnp.arange(_M), _K)
    cvec = jnp.stack(
        [cvx[:, :_K], cvy[:, :_K], cvz[:, :_K]], axis=-1).reshape(-1, 3)
    cd = cdp[:, :_K].reshape(-1)
    cvalid = cvp[:, :_K].reshape(-1).astype(bool)
    coarse_e = jnp.stack([src, dst])

    return (node_dst_idx, bi_e, edge_vec, d, valid, coarse_e, cvec, cd, cvalid)


# speedup vs baseline: 1.5109x; 1.5109x over previous
import functools

import jax
import jax.numpy as jnp
from jax.experimental import pallas as pl
from jax.experimental.pallas import tpu as pltpu

_N = 8192
_M = 4096
_K = 32
_EPS = 1e-3
_BIG = 1e30


def _fps_body(xx_ref, xy_ref, xz_ref, xsx_ref, xsy_ref, xsz_ref, sel_ref):
    xx = xx_ref[...]
    xy = xy_ref[...]
    xz = xz_ref[...]
    shape = xx.shape
    idx = (jax.lax.broadcasted_iota(jnp.int32, shape, 0) * 128
           + jax.lax.broadcasted_iota(jnp.int32, shape, 1))
    x0x = xsx_ref[0]
    x0y = xsy_ref[0]
    x0z = xsz_ref[0]
    mind = ((xx - x0x) ** 2 + (xz - x0z) ** 2) + (xy - x0y) ** 2
    sel_ref[0] = 0

    def body(i, mind):
        maxv = jnp.max(mind)
        nxt = jnp.min(jnp.where(mind == maxv, idx, _N))
        cx = xsx_ref[nxt]
        cy = xsy_ref[nxt]
        cz = xsz_ref[nxt]
        d = ((xx - cx) ** 2 + (xz - cz) ** 2) + (xy - cy) ** 2
        sel_ref[i] = nxt
        return jnp.minimum(mind, d)

    jax.lax.fori_loop(1, _M, body, mind)


def _fps(x_cor):
    xx = x_cor[:, 0].reshape(64, 128)
    xy = x_cor[:, 1].reshape(64, 128)
    xz = x_cor[:, 2].reshape(64, 128)
    sel = pl.pallas_call(
        _fps_body,
        in_specs=[
            pl.BlockSpec(memory_space=pltpu.VMEM),
            pl.BlockSpec(memory_space=pltpu.VMEM),
            pl.BlockSpec(memory_space=pltpu.VMEM),
            pl.BlockSpec(memory_space=pltpu.SMEM),
            pl.BlockSpec(memory_space=pltpu.SMEM),
            pl.BlockSpec(memory_space=pltpu.SMEM),
        ],
        out_shape=jax.ShapeDtypeStruct((_M,), jnp.int32),
        out_specs=pl.BlockSpec(memory_space=pltpu.SMEM),
    )(xx, xy, xz, x_cor[:, 0], x_cor[:, 1], x_cor[:, 2])
    return sel


def _knn_body(nsteps, diag, bm, nc,
              yblk_ref, xt_ref,
              idx_out, vx_out, vy_out, vz_out, d_out, valid_out,
              d2_ref):
    yb = yblk_ref[...]
    yx = yb[:, 0:1]
    yy = yb[:, 1:2]
    yz = yb[:, 2:3]
    xt = xt_ref[...]
    xxr = xt[0:1, :]
    xyr = xt[1:2, :]
    xzr = xt[2:3, :]
    sa = yx * yx + yy * yy + yz * yz
    sb = xxr * xxr + xyr * xyr + xzr * xzr
    mm = jnp.dot(yb, xt, preferred_element_type=jnp.float32)
    d2 = sa + sb - 2.0 * mm
    col = jax.lax.broadcasted_iota(jnp.int32, (bm, nc), 1)
    if diag:
        rowi = (pl.program_id(0) * bm
                + jax.lax.broadcasted_iota(jnp.int32, (bm, nc), 0))
        d2 = jnp.where(col == rowi, jnp.inf, d2)
    d2_ref[...] = d2

    lane = jax.lax.broadcasted_iota(jnp.int32, (bm, 64), 1)
    zf = jnp.zeros((bm, 64), jnp.float32)
    zi = jnp.zeros((bm, 64), jnp.int32)

    def step(k, accs):
        acc_i, acc_x, acc_y, acc_z = accs
        d2c = d2_ref[...]
        mv = jnp.min(d2c, axis=1, keepdims=True)
        nidx = jnp.min(jnp.where(d2c == mv, col, nc), axis=1, keepdims=True)
        oh = col == nidx
        d2_ref[...] = jnp.where(oh, jnp.inf, d2c)
        cx = jnp.sum(jnp.where(oh, xxr, 0.0), axis=1, keepdims=True)
        cy = jnp.sum(jnp.where(oh, xyr, 0.0), axis=1, keepdims=True)
        cz = jnp.sum(jnp.where(oh, xzr, 0.0), axis=1, keepdims=True)
        acc_i = jnp.where(lane == k, nidx, acc_i)
        acc_x = jnp.where(lane == k, cx - yx, acc_x)
        acc_y = jnp.where(lane == k, cy - yy, acc_y)
        acc_z = jnp.where(lane == k, cz - yz, acc_z)
        return acc_i, acc_x, acc_y, acc_z

    acc_i, acc_x, acc_y, acc_z = jax.lax.fori_loop(
        0, nsteps, step, (zi, zf, zf, zf))

    dd = jnp.sqrt(acc_x * acc_x + acc_y * acc_y + acc_z * acc_z)
    vmask = dd > _EPS
    idx_out[...] = acc_i
    vx_out[...] = jnp.where(vmask, acc_x, 0.0)
    vy_out[...] = jnp.where(vmask, acc_y, 0.0)
    vz_out[...] = jnp.where(vmask, acc_z, 0.0)
    d_out[...] = jnp.where(vmask, dd, 0.0)
    valid_out[...] = vmask.astype(jnp.int32)


def _knn(y, xt, nsteps, diag, bm):
    mq = y.shape[0]
    nc = xt.shape[1]
    f32 = jnp.float32
    body = functools.partial(_knn_body, nsteps, diag, bm, nc)
    outs = pl.pallas_call(
        body,
        grid=(mq // bm,),
        in_specs=[
            pl.BlockSpec((bm, 3), lambda i: (i, 0)),
            pl.BlockSpec((3, nc), lambda i: (0, 0)),
        ],
        out_specs=[pl.BlockSpec((bm, 64), lambda i: (i, 0))] * 6,
        out_shape=[
            jax.ShapeDtypeStruct((mq, 64), jnp.int32),
            jax.ShapeDtypeStruct((mq, 64), f32),
            jax.ShapeDtypeStruct((mq, 64), f32),
            jax.ShapeDtypeStruct((mq, 64), f32),
            jax.ShapeDtypeStruct((mq, 64), f32),
            jax.ShapeDtypeStruct((mq, 64), jnp.int32),
        ],
        scratch_shapes=[pltpu.VMEM((bm, nc), f32)],
    )(y, xt)
    return outs


def kernel(x_cor, piece_index, L):
    del piece_index, L
    sel = _fps(x_cor)
    node_dst_idx = jnp.sort(sel)
    y = x_cor[node_dst_idx]
    xt = x_cor.T
    yt = y.T

    kb = _K + 1
    nbr_i, vx, vy, vz, dpad, validpad = _knn(y, xt, kb, False, 256)
    x_idx = nbr_i[:, :kb].reshape(-1)
    y_idx = jnp.repeat(jnp.arange(_M), kb)
    edge_vec = jnp.stack(
        [vx[:, :kb], vy[:, :kb], vz[:, :kb]], axis=-1).reshape(-1, 3)
    d = dpad[:, :kb].reshape(-1)
    valid = validpad[:, :kb].reshape(-1).astype(bool)
    bi_e = jnp.stack([x_idx, y_idx])

    ci, cvx, cvy, cvz, cdp, cvp = _knn(y, yt, _K, True, 256)
    src = ci[:, :_K].reshape(-1)
    dst = jnp.repeat(jnp.arange(_M), _K)
    cvec = jnp.stack(
        [cvx[:, :_K], cvy[:, :_K], cvz[:, :_K]], axis=-1).reshape(-1, 3)
    cd = cdp[:, :_K].reshape(-1)
    cvalid = cvp[:, :_K].reshape(-1).astype(bool)
    coarse_e = jnp.stack([src, dst])

    return (node_dst_idx, bi_e, edge_vec, d, valid, coarse_e, cvec, cd, cvalid)


# speedup vs baseline: 1.5111x; 1.0001x over previous
import functools

import jax
import jax.numpy as jnp
from jax.experimental import pallas as pl
from jax.experimental.pallas import tpu as pltpu

_N = 8192
_M = 4096
_K = 32
_EPS = 1e-3
_BIG = 1e30


def _fps_body(xx_ref, xy_ref, xz_ref, xsx_ref, xsy_ref, xsz_ref, sel_ref):
    xx = xx_ref[...]
    xy = xy_ref[...]
    xz = xz_ref[...]
    shape = xx.shape
    idx = (jax.lax.broadcasted_iota(jnp.int32, shape, 0) * 128
           + jax.lax.broadcasted_iota(jnp.int32, shape, 1))
    x0x = xsx_ref[0]
    x0y = xsy_ref[0]
    x0z = xsz_ref[0]
    mind = ((xx - x0x) ** 2 + (xz - x0z) ** 2) + (xy - x0y) ** 2
    sel_ref[0] = 0

    def body(i, mind):
        maxv = jnp.max(mind)
        nxt = jnp.min(jnp.where(mind == maxv, idx, _N))
        cx = xsx_ref[nxt]
        cy = xsy_ref[nxt]
        cz = xsz_ref[nxt]
        d = ((xx - cx) ** 2 + (xz - cz) ** 2) + (xy - cy) ** 2
        sel_ref[i] = nxt
        return jnp.minimum(mind, d)

    jax.lax.fori_loop(1, _M, body, mind)


def _fps(x_cor):
    xx = x_cor[:, 0].reshape(64, 128)
    xy = x_cor[:, 1].reshape(64, 128)
    xz = x_cor[:, 2].reshape(64, 128)
    sel = pl.pallas_call(
        _fps_body,
        in_specs=[
            pl.BlockSpec(memory_space=pltpu.VMEM),
            pl.BlockSpec(memory_space=pltpu.VMEM),
            pl.BlockSpec(memory_space=pltpu.VMEM),
            pl.BlockSpec(memory_space=pltpu.SMEM),
            pl.BlockSpec(memory_space=pltpu.SMEM),
            pl.BlockSpec(memory_space=pltpu.SMEM),
        ],
        out_shape=jax.ShapeDtypeStruct((_M,), jnp.int32),
        out_specs=pl.BlockSpec(memory_space=pltpu.SMEM),
    )(xx, xy, xz, x_cor[:, 0], x_cor[:, 1], x_cor[:, 2])
    return sel


def _knn_body(nsteps, diag, bm, nc,
              yblk_ref, xt_ref,
              idx_out, vx_out, vy_out, vz_out, d_out, valid_out,
              d2_ref):
    yb = yblk_ref[...]
    yx = yb[:, 0:1]
    yy = yb[:, 1:2]
    yz = yb[:, 2:3]
    xt = xt_ref[...]
    xxr = xt[0:1, :]
    xyr = xt[1:2, :]
    xzr = xt[2:3, :]
    sa = (yx * yx + yz * yz) + yy * yy
    sb = (xxr * xxr + xzr * xzr) + xyr * xyr
    mm = jnp.dot(yb, xt, preferred_element_type=jnp.float32)
    d2 = sa + sb - 2.0 * mm
    col = jax.lax.broadcasted_iota(jnp.int32, (bm, nc), 1)
    if diag:
        rowi = (pl.program_id(0) * bm
                + jax.lax.broadcasted_iota(jnp.int32, (bm, nc), 0))
        d2 = jnp.where(col == rowi, jnp.inf, d2)
    d2_ref[...] = d2

    lane = jax.lax.broadcasted_iota(jnp.int32, (bm, 64), 1)
    zf = jnp.zeros((bm, 64), jnp.float32)
    zi = jnp.zeros((bm, 64), jnp.int32)

    def step(k, accs):
        acc_i, acc_x, acc_y, acc_z = accs
        d2c = d2_ref[...]
        mv = jnp.min(d2c, axis=1, keepdims=True)
        nidx = jnp.min(jnp.where(d2c == mv, col, nc), axis=1, keepdims=True)
        oh = col == nidx
        d2_ref[...] = jnp.where(oh, jnp.inf, d2c)
        cx = jnp.sum(jnp.where(oh, xxr, 0.0), axis=1, keepdims=True)
        cy = jnp.sum(jnp.where(oh, xyr, 0.0), axis=1, keepdims=True)
        cz = jnp.sum(jnp.where(oh, xzr, 0.0), axis=1, keepdims=True)
        acc_i = jnp.where(lane == k, nidx, acc_i)
        acc_x = jnp.where(lane == k, cx - yx, acc_x)
        acc_y = jnp.where(lane == k, cy - yy, acc_y)
        acc_z = jnp.where(lane == k, cz - yz, acc_z)
        return acc_i, acc_x, acc_y, acc_z

    acc_i, acc_x, acc_y, acc_z = jax.lax.fori_loop(
        0, nsteps, step, (zi, zf, zf, zf))

    dd = jnp.sqrt((acc_x * acc_x + acc_z * acc_z) + acc_y * acc_y)
    vmask = dd > _EPS
    idx_out[...] = acc_i
    vx_out[...] = jnp.where(vmask, acc_x, 0.0)
    vy_out[...] = jnp.where(vmask, acc_y, 0.0)
    vz_out[...] = jnp.where(vmask, acc_z, 0.0)
    d_out[...] = jnp.where(vmask, dd, 0.0)
    valid_out[...] = vmask.astype(jnp.int32)


def _knn(y, xt, nsteps, diag, bm):
    mq = y.shape[0]
    nc = xt.shape[1]
    f32 = jnp.float32
    body = functools.partial(_knn_body, nsteps, diag, bm, nc)
    outs = pl.pallas_call(
        body,
        grid=(mq // bm,),
        in_specs=[
            pl.BlockSpec((bm, 3), lambda i: (i, 0)),
            pl.BlockSpec((3, nc), lambda i: (0, 0)),
        ],
        out_specs=[pl.BlockSpec((bm, 64), lambda i: (i, 0))] * 6,
        out_shape=[
            jax.ShapeDtypeStruct((mq, 64), jnp.int32),
            jax.ShapeDtypeStruct((mq, 64), f32),
            jax.ShapeDtypeStruct((mq, 64), f32),
            jax.ShapeDtypeStruct((mq, 64), f32),
            jax.ShapeDtypeStruct((mq, 64), f32),
            jax.ShapeDtypeStruct((mq, 64), jnp.int32),
        ],
        scratch_shapes=[pltpu.VMEM((bm, nc), f32)],
    )(y, xt)
    return outs


def kernel(x_cor, piece_index, L):
    del piece_index, L
    sel = _fps(x_cor)
    node_dst_idx = jnp.sort(sel)
    y = x_cor[node_dst_idx]
    xt = x_cor.T
    yt = y.T

    kb = _K + 1
    nbr_i, vx, vy, vz, dpad, validpad = _knn(y, xt, kb, False, 256)
    x_idx = nbr_i[:, :kb].reshape(-1)
    y_idx = jnp.repeat(jnp.arange(_M), kb)
    edge_vec = jnp.stack(
        [vx[:, :kb], vy[:, :kb], vz[:, :kb]], axis=-1).reshape(-1, 3)
    d = dpad[:, :kb].reshape(-1)
    valid = validpad[:, :kb].reshape(-1).astype(bool)
    bi_e = jnp.stack([x_idx, y_idx])

    ci, cvx, cvy, cvz, cdp, cvp = _knn(y, yt, _K, True, 256)
    src = ci[:, :_K].reshape(-1)
    dst = jnp.repeat(jnp.arange(_M), _K)
    cvec = jnp.stack(
        [cvx[:, :_K], cvy[:, :_K], cvz[:, :_K]], axis=-1).reshape(-1, 3)
    cd = cdp[:, :_K].reshape(-1)
    cvalid = cvp[:, :_K].reshape(-1).astype(bool)
    coarse_e = jnp.stack([src, dst])

    return (node_dst_idx, bi_e, edge_vec, d, valid, coarse_e, cvec, cd, cvalid)
